# baseline (speedup 1.0000x reference)
import functools
import math

import jax
import jax.numpy as jnp
from jax import lax
from jax.experimental import pallas as pl
from jax.experimental.pallas import tpu as pltpu

N = 50000
E = 800000
ETOT = E + N
FEAT = 64
H1, C1 = 3, 64
H2, C2 = 2, 32
DEC = 256

TILE = 256
NP = ((N + TILE - 1) // TILE) * TILE
NT = NP // TILE

_SQRT2 = math.sqrt(2.0)


def _gelu(v):
    return 0.5 * v * (1.0 + lax.erf(v / _SQRT2))


def _ln_rows(v, g, b, width):
    mu = jnp.mean(v, axis=-1, keepdims=True)
    var = jnp.mean((v - mu) ** 2, axis=-1, keepdims=True)
    return (v - mu) / jnp.sqrt(var + 1e-5) * g + b


def _pre_body(xc, expW, lng, lnb, pert, W1p, As, Ad,
              h1_o, as_o, ad_o, asmax_o, admax_o):
    i = pl.program_id(0)
    x = xc[...]
    v = x * expW[...]
    feat = _gelu(_ln_rows(v, lng[...], lnb[...], FEAT)) + pert[...]
    h1 = jnp.dot(feat, W1p[...], preferred_element_type=jnp.float32)
    h1_o[...] = h1
    a_s = jnp.dot(h1, As[...], preferred_element_type=jnp.float32, precision=lax.Precision.HIGHEST)
    a_d = jnp.dot(h1, Ad[...], preferred_element_type=jnp.float32, precision=lax.Precision.HIGHEST)
    as_o[...] = a_s
    ad_o[...] = a_d
    asm = jnp.max(a_s.reshape(TILE // 8, 8, 128), axis=0)
    adm = jnp.max(a_d.reshape(TILE // 8, 8, 128), axis=0)

    @pl.when(i == 0)
    def _():
        asmax_o[...] = jnp.full((8, 128), -jnp.inf, jnp.float32)
        admax_o[...] = jnp.full((8, 128), -jnp.inf, jnp.float32)

    asmax_o[...] = jnp.maximum(asmax_o[...], asm)
    admax_o[...] = jnp.maximum(admax_o[...], adm)


def _run_pre(xp, expW, lng, lnb, pert, W1p, As, Ad):
    wspec = lambda shape: pl.BlockSpec(shape, lambda i: (0, 0))
    return pl.pallas_call(
        _pre_body,
        grid=(NT,),
        in_specs=[
            pl.BlockSpec((TILE, 1), lambda i: (i, 0)),
            wspec((1, FEAT)), wspec((1, FEAT)), wspec((1, FEAT)),
            wspec((1, FEAT)), wspec((FEAT, 256)), wspec((256, 128)),
            wspec((256, 128)),
        ],
        out_specs=[
            pl.BlockSpec((TILE, 256), lambda i: (i, 0)),
            pl.BlockSpec((TILE, 128), lambda i: (i, 0)),
            pl.BlockSpec((TILE, 128), lambda i: (i, 0)),
            pl.BlockSpec((8, 128), lambda i: (0, 0)),
            pl.BlockSpec((8, 128), lambda i: (0, 0)),
        ],
        out_shape=[
            jax.ShapeDtypeStruct((NP, 256), jnp.float32),
            jax.ShapeDtypeStruct((NP, 128), jnp.float32),
            jax.ShapeDtypeStruct((NP, 128), jnp.float32),
            jax.ShapeDtypeStruct((8, 128), jnp.float32),
            jax.ShapeDtypeStruct((8, 128), jnp.float32),
        ],
        compiler_params=pltpu.CompilerParams(
            dimension_semantics=("arbitrary",)),
    )(xp, expW, lng, lnb, pert, W1p, As, Ad)


def _mid_body(acc, den, bias, Exp1, W2p, As, Ad,
              h2_o, as_o, ad_o, asmax_o, admax_o):
    i = pl.program_id(0)
    r = 1.0 / (den[...] + 1e-16)
    rbig = jnp.dot(r, Exp1[...], preferred_element_type=jnp.float32, precision=lax.Precision.HIGHEST)
    out1 = acc[...] * rbig + bias[...]
    h = jnp.where(out1 > 0, out1, jnp.exp(out1) - 1.0)
    h2 = jnp.dot(h, W2p[...], preferred_element_type=jnp.float32)
    h2_o[...] = h2
    a_s = jnp.dot(h2, As[...], preferred_element_type=jnp.float32, precision=lax.Precision.HIGHEST)
    a_d = jnp.dot(h2, Ad[...], preferred_element_type=jnp.float32, precision=lax.Precision.HIGHEST)
    as_o[...] = a_s
    ad_o[...] = a_d
    asm = jnp.max(a_s.reshape(TILE // 8, 8, 128), axis=0)
    adm = jnp.max(a_d.reshape(TILE // 8, 8, 128), axis=0)

    @pl.when(i == 0)
    def _():
        asmax_o[...] = jnp.full((8, 128), -jnp.inf, jnp.float32)
        admax_o[...] = jnp.full((8, 128), -jnp.inf, jnp.float32)

    asmax_o[...] = jnp.maximum(asmax_o[...], asm)
    admax_o[...] = jnp.maximum(admax_o[...], adm)


def _run_mid(acc, den, bias, Exp1, W2p, As, Ad):
    wspec = lambda shape: pl.BlockSpec(shape, lambda i: (0, 0))
    return pl.pallas_call(
        _mid_body,
        grid=(NT,),
        in_specs=[
            pl.BlockSpec((TILE, 256), lambda i: (i, 0)),
            pl.BlockSpec((TILE, 128), lambda i: (i, 0)),
            wspec((1, 256)), wspec((128, 256)), wspec((256, 128)),
            wspec((128, 128)), wspec((128, 128)),
        ],
        out_specs=[
            pl.BlockSpec((TILE, 128), lambda i: (i, 0)),
            pl.BlockSpec((TILE, 128), lambda i: (i, 0)),
            pl.BlockSpec((TILE, 128), lambda i: (i, 0)),
            pl.BlockSpec((8, 128), lambda i: (0, 0)),
            pl.BlockSpec((8, 128), lambda i: (0, 0)),
        ],
        out_shape=[
            jax.ShapeDtypeStruct((NP, 128), jnp.float32),
            jax.ShapeDtypeStruct((NP, 128), jnp.float32),
            jax.ShapeDtypeStruct((NP, 128), jnp.float32),
            jax.ShapeDtypeStruct((8, 128), jnp.float32),
            jax.ShapeDtypeStruct((8, 128), jnp.float32),
        ],
        compiler_params=pltpu.CompilerParams(
            dimension_semantics=("arbitrary",)),
    )(acc, den, bias, Exp1, W2p, As, Ad)


def _dec_body(acc, den, bias, Exp2, d1W, d1b, l1g, l1b,
              d2W, d2b, l2g, l2b, d3W, d3b, y_o):
    r = 1.0 / (den[...] + 1e-16)
    rbig = jnp.dot(r, Exp2[...], preferred_element_type=jnp.float32, precision=lax.Precision.HIGHEST)
    out2 = acc[...] * rbig + bias[...]
    z = jnp.dot(out2, d1W[...], preferred_element_type=jnp.float32) + d1b[...]
    z = _gelu(_ln_rows(z, l1g[...], l1b[...], DEC))
    z = jnp.dot(z, d2W[...], preferred_element_type=jnp.float32) + d2b[...]
    z = _gelu(_ln_rows(z, l2g[...], l2b[...], DEC))
    y = jnp.dot(z, d3W[...], preferred_element_type=jnp.float32) + d3b[...]
    y_o[...] = y


def _run_dec(acc, den, bias, Exp2, d1W, d1b, l1g, l1b,
             d2W, d2b, l2g, l2b, d3W, d3b):
    wspec = lambda shape: pl.BlockSpec(shape, lambda i: (0, 0))
    return pl.pallas_call(
        _dec_body,
        grid=(NT,),
        in_specs=[
            pl.BlockSpec((TILE, 128), lambda i: (i, 0)),
            pl.BlockSpec((TILE, 128), lambda i: (i, 0)),
            wspec((1, 128)), wspec((128, 128)),
            wspec((128, DEC)), wspec((1, DEC)), wspec((1, DEC)),
            wspec((1, DEC)), wspec((DEC, DEC)), wspec((1, DEC)),
            wspec((1, DEC)), wspec((1, DEC)), wspec((DEC, 128)),
            wspec((1, 128)),
        ],
        out_specs=[pl.BlockSpec((TILE, 128), lambda i: (i, 0))],
        out_shape=[jax.ShapeDtypeStruct((NP, 128), jnp.float32)],
        compiler_params=pltpu.CompilerParams(
            dimension_semantics=("arbitrary",)),
    )(acc, den, bias, Exp2, d1W, d1b, l1g, l1b,
      d2W, d2b, l2g, l2b, d3W, d3b)[0]


def _edge_phase(h_nodes, a_s, a_d, M, src, dst, heads, ch):
    hc = heads * ch
    alpha = a_s[src, :heads] + a_d[dst, :heads]
    alpha = jnp.where(alpha > 0, alpha, 0.2 * alpha)
    ex = jnp.exp(alpha - M[:heads])
    denom = jax.ops.segment_sum(ex, dst, num_segments=N)
    msg = (h_nodes[src, :hc].reshape(-1, heads, ch)
           * ex[:, :, None]).reshape(-1, hc)
    accum = jax.ops.segment_sum(msg, dst, num_segments=N)
    return accum, denom


def _lrelu(x):
    return jnp.where(x > 0, x, 0.2 * x)


def kernel(x_ctrl, edge_index, pert_id, exp_W, exp_b, exp_ln_g, exp_ln_b,
           pert_table, W1, att_src1, att_dst1, bias1,
           W2, att_src2, att_dst2, bias2,
           d1_W, d1_b, ln1_g, ln1_b, d2_W, d2_b, ln2_g, ln2_b, d3_W, d3_b):
    f32 = jnp.float32
    loop = jnp.arange(N, dtype=edge_index.dtype)
    src = jnp.concatenate([edge_index[0], loop])
    dst = jnp.concatenate([edge_index[1], loop])

    xp = jnp.pad(x_ctrl, (0, NP - N)).reshape(NP, 1)
    pert = pert_table[pert_id[0:1]]
    W1p = jnp.pad(W1, ((0, 0), (0, 256 - H1 * C1)))
    r1 = jnp.arange(H1 * C1)
    As1 = jnp.zeros((256, 128), f32).at[r1, r1 // C1].set(att_src1.reshape(-1))
    Ad1 = jnp.zeros((256, 128), f32).at[r1, r1 // C1].set(att_dst1.reshape(-1))
    Exp1 = jnp.zeros((128, 256), f32).at[r1 // C1, r1].set(1.0)
    b1 = jnp.pad(bias1, (0, 256 - H1 * C1)).reshape(1, 256)
    W2p = jnp.pad(W2, ((0, 256 - H1 * C1), (0, 128 - H2 * C2)))
    r2 = jnp.arange(H2 * C2)
    As2 = jnp.zeros((128, 128), f32).at[r2, r2 // C2].set(att_src2.reshape(-1))
    Ad2 = jnp.zeros((128, 128), f32).at[r2, r2 // C2].set(att_dst2.reshape(-1))
    Exp2 = jnp.zeros((128, 128), f32).at[r2 // C2, r2].set(1.0)
    b2 = jnp.pad(bias2, (0, 128 - H2 * C2)).reshape(1, 128)
    d1Wp = jnp.pad(d1_W, ((0, 128 - H2 * C2), (0, 0)))
    d3Wp = jnp.pad(d3_W, ((0, 0), (0, 127)))
    d3bp = jnp.pad(d3_b, (0, 127)).reshape(1, 128)

    h1, a1s, a1d, a1sm, a1dm = _run_pre(
        xp, exp_W, exp_ln_g.reshape(1, -1), exp_ln_b.reshape(1, -1),
        pert, W1p, As1, Ad1)
    M1 = _lrelu(jnp.max(a1sm, axis=0) + jnp.max(a1dm, axis=0))

    acc1, den1 = _edge_phase(h1, a1s, a1d, M1, src, dst, H1, C1)
    acc1 = jnp.pad(acc1, ((0, NP - N), (0, 256 - H1 * C1)))
    den1 = jnp.pad(den1, ((0, NP - N), (0, 128 - H1)), constant_values=1.0)

    h2, a2s, a2d, a2sm, a2dm = _run_mid(acc1, den1, b1, Exp1, W2p, As2, Ad2)
    M2 = _lrelu(jnp.max(a2sm, axis=0) + jnp.max(a2dm, axis=0))

    acc2, den2 = _edge_phase(h2, a2s, a2d, M2, src, dst, H2, C2)
    acc2 = jnp.pad(acc2, ((0, NP - N), (0, 128 - H2 * C2)))
    den2 = jnp.pad(den2, ((0, NP - N), (0, 128 - H2)), constant_values=1.0)

    y = _run_dec(acc2, den2, b2, Exp2, d1Wp, d1_b.reshape(1, -1),
                 ln1_g.reshape(1, -1), ln1_b.reshape(1, -1),
                 d2_W, d2_b.reshape(1, -1), ln2_g.reshape(1, -1),
                 ln2_b.reshape(1, -1), d3Wp, d3bp)
    return y[:N, 0]


# speedup vs baseline: 10.7657x; 10.7657x over previous
import functools
import math

import jax
import jax.numpy as jnp
from jax import lax
from jax.experimental import pallas as pl
from jax.experimental.pallas import tpu as pltpu

N = 50000
E = 800000
ETOT = E + N
FEAT = 64
H1, C1 = 3, 64
H2, C2 = 2, 32
DEC = 256

TILE = 256
NP = ((N + TILE - 1) // TILE) * TILE
NT = NP // TILE

_SQRT2 = math.sqrt(2.0)


def _gelu(v):
    return 0.5 * v * (1.0 + lax.erf(v / _SQRT2))


def _ln_rows(v, g, b, width):
    mu = jnp.mean(v, axis=-1, keepdims=True)
    var = jnp.mean((v - mu) ** 2, axis=-1, keepdims=True)
    return (v - mu) / jnp.sqrt(var + 1e-5) * g + b


def _pre_body(xc, expW, lng, lnb, pert, W1p, As, Ad,
              h1_o, as_o, ad_o, asmax_o, admax_o):
    i = pl.program_id(0)
    x = xc[...]
    v = x * expW[...]
    feat = _gelu(_ln_rows(v, lng[...], lnb[...], FEAT)) + pert[...]
    h1 = jnp.dot(feat, W1p[...], preferred_element_type=jnp.float32)
    h1_o[...] = h1
    a_s = jnp.dot(h1, As[...], preferred_element_type=jnp.float32, precision=lax.Precision.HIGHEST)
    a_d = jnp.dot(h1, Ad[...], preferred_element_type=jnp.float32, precision=lax.Precision.HIGHEST)
    as_o[...] = a_s
    ad_o[...] = a_d
    asm = jnp.max(a_s.reshape(TILE // 8, 8, 128), axis=0)
    adm = jnp.max(a_d.reshape(TILE // 8, 8, 128), axis=0)

    @pl.when(i == 0)
    def _():
        asmax_o[...] = jnp.full((8, 128), -jnp.inf, jnp.float32)
        admax_o[...] = jnp.full((8, 128), -jnp.inf, jnp.float32)

    asmax_o[...] = jnp.maximum(asmax_o[...], asm)
    admax_o[...] = jnp.maximum(admax_o[...], adm)


def _run_pre(xp, expW, lng, lnb, pert, W1p, As, Ad):
    wspec = lambda shape: pl.BlockSpec(shape, lambda i: (0, 0))
    return pl.pallas_call(
        _pre_body,
        grid=(NT,),
        in_specs=[
            pl.BlockSpec((TILE, 1), lambda i: (i, 0)),
            wspec((1, FEAT)), wspec((1, FEAT)), wspec((1, FEAT)),
            wspec((1, FEAT)), wspec((FEAT, 256)), wspec((256, 128)),
            wspec((256, 128)),
        ],
        out_specs=[
            pl.BlockSpec((TILE, 256), lambda i: (i, 0)),
            pl.BlockSpec((TILE, 128), lambda i: (i, 0)),
            pl.BlockSpec((TILE, 128), lambda i: (i, 0)),
            pl.BlockSpec((8, 128), lambda i: (0, 0)),
            pl.BlockSpec((8, 128), lambda i: (0, 0)),
        ],
        out_shape=[
            jax.ShapeDtypeStruct((NP, 256), jnp.float32),
            jax.ShapeDtypeStruct((NP, 128), jnp.float32),
            jax.ShapeDtypeStruct((NP, 128), jnp.float32),
            jax.ShapeDtypeStruct((8, 128), jnp.float32),
            jax.ShapeDtypeStruct((8, 128), jnp.float32),
        ],
        compiler_params=pltpu.CompilerParams(
            dimension_semantics=("arbitrary",)),
    )(xp, expW, lng, lnb, pert, W1p, As, Ad)


def _mid_body(acc, den, bias, Exp1, W2p, As, Ad,
              h2_o, as_o, ad_o, asmax_o, admax_o):
    i = pl.program_id(0)
    r = 1.0 / (den[...] + 1e-16)
    rbig = jnp.dot(r, Exp1[...], preferred_element_type=jnp.float32, precision=lax.Precision.HIGHEST)
    out1 = acc[...] * rbig + bias[...]
    h = jnp.where(out1 > 0, out1, jnp.exp(out1) - 1.0)
    h2 = jnp.dot(h, W2p[...], preferred_element_type=jnp.float32)
    h2_o[...] = h2
    a_s = jnp.dot(h2, As[...], preferred_element_type=jnp.float32, precision=lax.Precision.HIGHEST)
    a_d = jnp.dot(h2, Ad[...], preferred_element_type=jnp.float32, precision=lax.Precision.HIGHEST)
    as_o[...] = a_s
    ad_o[...] = a_d
    asm = jnp.max(a_s.reshape(TILE // 8, 8, 128), axis=0)
    adm = jnp.max(a_d.reshape(TILE // 8, 8, 128), axis=0)

    @pl.when(i == 0)
    def _():
        asmax_o[...] = jnp.full((8, 128), -jnp.inf, jnp.float32)
        admax_o[...] = jnp.full((8, 128), -jnp.inf, jnp.float32)

    asmax_o[...] = jnp.maximum(asmax_o[...], asm)
    admax_o[...] = jnp.maximum(admax_o[...], adm)


def _run_mid(acc, den, bias, Exp1, W2p, As, Ad):
    wspec = lambda shape: pl.BlockSpec(shape, lambda i: (0, 0))
    return pl.pallas_call(
        _mid_body,
        grid=(NT,),
        in_specs=[
            pl.BlockSpec((TILE, 256), lambda i: (i, 0)),
            pl.BlockSpec((TILE, 128), lambda i: (i, 0)),
            wspec((1, 256)), wspec((128, 256)), wspec((256, 128)),
            wspec((128, 128)), wspec((128, 128)),
        ],
        out_specs=[
            pl.BlockSpec((TILE, 128), lambda i: (i, 0)),
            pl.BlockSpec((TILE, 128), lambda i: (i, 0)),
            pl.BlockSpec((TILE, 128), lambda i: (i, 0)),
            pl.BlockSpec((8, 128), lambda i: (0, 0)),
            pl.BlockSpec((8, 128), lambda i: (0, 0)),
        ],
        out_shape=[
            jax.ShapeDtypeStruct((NP, 128), jnp.float32),
            jax.ShapeDtypeStruct((NP, 128), jnp.float32),
            jax.ShapeDtypeStruct((NP, 128), jnp.float32),
            jax.ShapeDtypeStruct((8, 128), jnp.float32),
            jax.ShapeDtypeStruct((8, 128), jnp.float32),
        ],
        compiler_params=pltpu.CompilerParams(
            dimension_semantics=("arbitrary",)),
    )(acc, den, bias, Exp1, W2p, As, Ad)


def _dec_body(acc, den, bias, Exp2, d1W, d1b, l1g, l1b,
              d2W, d2b, l2g, l2b, d3W, d3b, y_o):
    r = 1.0 / (den[...] + 1e-16)
    rbig = jnp.dot(r, Exp2[...], preferred_element_type=jnp.float32, precision=lax.Precision.HIGHEST)
    out2 = acc[...] * rbig + bias[...]
    z = jnp.dot(out2, d1W[...], preferred_element_type=jnp.float32) + d1b[...]
    z = _gelu(_ln_rows(z, l1g[...], l1b[...], DEC))
    z = jnp.dot(z, d2W[...], preferred_element_type=jnp.float32) + d2b[...]
    z = _gelu(_ln_rows(z, l2g[...], l2b[...], DEC))
    y = jnp.dot(z, d3W[...], preferred_element_type=jnp.float32) + d3b[...]
    y_o[...] = y


def _run_dec(acc, den, bias, Exp2, d1W, d1b, l1g, l1b,
             d2W, d2b, l2g, l2b, d3W, d3b):
    wspec = lambda shape: pl.BlockSpec(shape, lambda i: (0, 0))
    return pl.pallas_call(
        _dec_body,
        grid=(NT,),
        in_specs=[
            pl.BlockSpec((TILE, 128), lambda i: (i, 0)),
            pl.BlockSpec((TILE, 128), lambda i: (i, 0)),
            wspec((1, 128)), wspec((128, 128)),
            wspec((128, DEC)), wspec((1, DEC)), wspec((1, DEC)),
            wspec((1, DEC)), wspec((DEC, DEC)), wspec((1, DEC)),
            wspec((1, DEC)), wspec((1, DEC)), wspec((DEC, 128)),
            wspec((1, 128)),
        ],
        out_specs=[pl.BlockSpec((TILE, 128), lambda i: (i, 0))],
        out_shape=[jax.ShapeDtypeStruct((NP, 128), jnp.float32)],
        compiler_params=pltpu.CompilerParams(
            dimension_semantics=("arbitrary",)),
    )(acc, den, bias, Exp2, d1W, d1b, l1g, l1b,
      d2W, d2b, l2g, l2b, d3W, d3b)[0]


def _edge_phase(h_nodes, a_s, a_d, M, src, dst, heads, ch):
    hc = heads * ch
    hn = h_nodes[:N, :hc]
    asn = a_s[:N, :heads]
    adn = a_d[:N, :heads]
    alpha = asn[src] + adn[dst]
    alpha = jnp.where(alpha > 0, alpha, 0.2 * alpha)
    ex = jnp.exp(alpha - M[:heads])
    denom = jax.ops.segment_sum(ex, dst, num_segments=N)
    msg = (hn[src].reshape(-1, heads, ch)
           * ex[:, :, None]).reshape(-1, hc)
    accum = jax.ops.segment_sum(msg, dst, num_segments=N)
    return accum, denom


def _lrelu(x):
    return jnp.where(x > 0, x, 0.2 * x)


def kernel(x_ctrl, edge_index, pert_id, exp_W, exp_b, exp_ln_g, exp_ln_b,
           pert_table, W1, att_src1, att_dst1, bias1,
           W2, att_src2, att_dst2, bias2,
           d1_W, d1_b, ln1_g, ln1_b, d2_W, d2_b, ln2_g, ln2_b, d3_W, d3_b):
    f32 = jnp.float32
    loop = jnp.arange(N, dtype=edge_index.dtype)
    src = jnp.concatenate([edge_index[0], loop])
    dst = jnp.concatenate([edge_index[1], loop])

    xp = jnp.pad(x_ctrl, (0, NP - N)).reshape(NP, 1)
    pert = pert_table[pert_id[0:1]]
    W1p = jnp.pad(W1, ((0, 0), (0, 256 - H1 * C1)))
    r1 = jnp.arange(H1 * C1)
    As1 = jnp.zeros((256, 128), f32).at[r1, r1 // C1].set(att_src1.reshape(-1))
    Ad1 = jnp.zeros((256, 128), f32).at[r1, r1 // C1].set(att_dst1.reshape(-1))
    Exp1 = jnp.zeros((128, 256), f32).at[r1 // C1, r1].set(1.0)
    b1 = jnp.pad(bias1, (0, 256 - H1 * C1)).reshape(1, 256)
    W2p = jnp.pad(W2, ((0, 256 - H1 * C1), (0, 128 - H2 * C2)))
    r2 = jnp.arange(H2 * C2)
    As2 = jnp.zeros((128, 128), f32).at[r2, r2 // C2].set(att_src2.reshape(-1))
    Ad2 = jnp.zeros((128, 128), f32).at[r2, r2 // C2].set(att_dst2.reshape(-1))
    Exp2 = jnp.zeros((128, 128), f32).at[r2 // C2, r2].set(1.0)
    b2 = jnp.pad(bias2, (0, 128 - H2 * C2)).reshape(1, 128)
    d1Wp = jnp.pad(d1_W, ((0, 128 - H2 * C2), (0, 0)))
    d3Wp = jnp.pad(d3_W, ((0, 0), (0, 127)))
    d3bp = jnp.pad(d3_b, (0, 127)).reshape(1, 128)

    h1, a1s, a1d, a1sm, a1dm = _run_pre(
        xp, exp_W, exp_ln_g.reshape(1, -1), exp_ln_b.reshape(1, -1),
        pert, W1p, As1, Ad1)
    M1 = _lrelu(jnp.max(a1sm, axis=0) + jnp.max(a1dm, axis=0))

    acc1, den1 = _edge_phase(h1, a1s, a1d, M1, src, dst, H1, C1)
    acc1 = jnp.pad(acc1, ((0, NP - N), (0, 256 - H1 * C1)))
    den1 = jnp.pad(den1, ((0, NP - N), (0, 128 - H1)), constant_values=1.0)

    h2, a2s, a2d, a2sm, a2dm = _run_mid(acc1, den1, b1, Exp1, W2p, As2, Ad2)
    M2 = _lrelu(jnp.max(a2sm, axis=0) + jnp.max(a2dm, axis=0))

    acc2, den2 = _edge_phase(h2, a2s, a2d, M2, src, dst, H2, C2)
    acc2 = jnp.pad(acc2, ((0, NP - N), (0, 128 - H2 * C2)))
    den2 = jnp.pad(den2, ((0, NP - N), (0, 128 - H2)), constant_values=1.0)

    y = _run_dec(acc2, den2, b2, Exp2, d1Wp, d1_b.reshape(1, -1),
                 ln1_g.reshape(1, -1), ln1_b.reshape(1, -1),
                 d2_W, d2_b.reshape(1, -1), ln2_g.reshape(1, -1),
                 ln2_b.reshape(1, -1), d3Wp, d3bp)
    return y[:N, 0]
